# baseline (speedup 1.0000x reference)
import jax
import jax.numpy as jnp
from jax import lax
from jax.experimental import pallas as pl
from jax.experimental.pallas import tpu as pltpu
from jax.experimental.pallas import tpu_sc as plsc

B = 16384
NUM_ROWS = 100000
D_EMB = 48
D_U = 16
D_OUT = D_EMB + D_U
L = 16

PACK = 8
D_PACK = PACK * D_EMB
NUM_PACKED = NUM_ROWS // PACK

NC = 2
NS = 16
NW = NC * NS
BPW = B // NW
CHUNK = 64
NCHUNK = BPW // CHUNK
NSLOT = 2


def _emb_body(x_hbm, tq_hbm, unique_hbm, out_hbm, idx_v, qidx_v, rows_v,
              out_v, u16_v, gsem, usem, wsem):
    wid = lax.axis_index("s") * NC + lax.axis_index("c")
    base = wid * BPW

    for j in range(NCHUNK):
        pltpu.sync_copy(x_hbm.at[pl.ds(base + j * CHUNK, CHUNK)], idx_v.at[j])
        for s in range(CHUNK // L):
            iv = idx_v[j, pl.ds(s * L, L)]
            qidx_v[j, pl.ds(s * L, L)] = lax.shift_right_logical(iv, 3)

    pltpu.async_copy(unique_hbm, u16_v, usem).wait()
    uvec = u16_v[...]
    lane = lax.iota(jnp.int32, L)

    def start_gather(j):
        return pltpu.async_copy(
            tq_hbm.at[qidx_v.at[j]], rows_v.at[j % NSLOT], gsem
        )

    gathers = [start_gather(0), start_gather(1)]
    writes = []
    for j in range(NCHUNK):
        slot = j % NSLOT
        gathers[j].wait()
        if j >= NSLOT:
            writes[j - NSLOT].wait()

        rows_slot = rows_v.at[slot]
        out_slot = out_v.at[slot]
        for blk in range(CHUNK // L):
            rowids = lane + (blk * L)
            iv = idx_v[j, pl.ds(blk * L, L)]
            ov = (iv & 7) * 48
            for c in range(D_EMB):
                vals = plsc.load_gather(rows_slot, [rowids, ov + c])
                plsc.store_scatter(
                    out_slot, [rowids, lax.full((L,), c, jnp.int32)], vals
                )

        def fill_u(i, carry):
            out_slot[i, pl.ds(D_EMB, D_U)] = uvec
            return carry

        lax.fori_loop(0, CHUNK, fill_u, 0, unroll=8)

        writes.append(
            pltpu.async_copy(
                out_slot,
                out_hbm.at[pl.ds(base + j * CHUNK, CHUNK)],
                wsem,
            )
        )
        if j + NSLOT < NCHUNK:
            gathers.append(start_gather(j + NSLOT))

    for w in writes[-NSLOT:]:
        w.wait()


_emb_call = pl.kernel(
    _emb_body,
    mesh=plsc.VectorSubcoreMesh(core_axis_name="c", subcore_axis_name="s"),
    out_type=jax.ShapeDtypeStruct((B, D_OUT), jnp.float32),
    compiler_params=pltpu.CompilerParams(
        use_tc_tiling_on_sc=False, needs_layout_passes=False
    ),
    scratch_types=[
        pltpu.VMEM((NCHUNK, CHUNK), jnp.int32),
        pltpu.VMEM((NCHUNK, CHUNK), jnp.int32),
        pltpu.VMEM((NSLOT, CHUNK, D_PACK), jnp.float32),
        pltpu.VMEM((NSLOT, CHUNK, D_OUT), jnp.float32),
        pltpu.VMEM((D_U,), jnp.float32),
        pltpu.SemaphoreType.DMA,
        pltpu.SemaphoreType.DMA,
        pltpu.SemaphoreType.DMA,
    ],
)


def kernel(x, table, unique):
    return _emb_call(x, table.reshape(NUM_PACKED, D_PACK), unique)


# speedup vs baseline: 1.1099x; 1.1099x over previous
import jax
import jax.numpy as jnp
from jax import lax
from jax.experimental import pallas as pl
from jax.experimental.pallas import tpu as pltpu
from jax.experimental.pallas import tpu_sc as plsc

B = 16384
NUM_ROWS = 100000
D_EMB = 48
D_U = 16
D_OUT = D_EMB + D_U
L = 16

PACK = 8
D_PACK = PACK * D_EMB
NUM_PACKED = NUM_ROWS // PACK

NC = 2
NS = 16
NW = NC * NS
BPW = B // NW
CHUNK = 64
NCHUNK = BPW // CHUNK
NSLOT = 2


def _emb_body(x_hbm, tq_hbm, unique_hbm, out_hbm, idx_v, qidx_v, rows_v,
              out_v, u16_v, gsem, usem, wsem):
    wid = lax.axis_index("s") * NC + lax.axis_index("c")
    base = wid * BPW

    for j in range(NCHUNK):
        pltpu.sync_copy(x_hbm.at[pl.ds(base + j * CHUNK, CHUNK)], idx_v.at[j])
        for s in range(CHUNK // L):
            iv = idx_v[j, pl.ds(s * L, L)]
            qidx_v[j, pl.ds(s * L, L)] = lax.shift_right_logical(iv, 3)

    pltpu.async_copy(unique_hbm, u16_v, usem).wait()
    uvec = u16_v[...]
    lane = lax.iota(jnp.int32, L)

    def start_gather(j):
        return pltpu.async_copy(
            tq_hbm.at[qidx_v.at[j]], rows_v.at[j % NSLOT], gsem
        )

    gathers = [start_gather(0), start_gather(1)]
    writes = []
    for j in range(NCHUNK):
        slot = j % NSLOT
        gathers[j].wait()
        if j >= NSLOT:
            writes[j - NSLOT].wait()

        rows_slot = rows_v.at[slot]
        out_slot = out_v.at[slot]
        def extract(blk, carry):
            rowids = lane + blk * L
            iv = idx_v[j, pl.ds(blk * L, L)]
            ov = (iv & 7) * 48
            for c in range(D_EMB):
                vals = plsc.load_gather(rows_slot, [rowids, ov + c])
                plsc.store_scatter(
                    out_slot, [rowids, lax.full((L,), c, jnp.int32)], vals
                )
            return carry

        lax.fori_loop(0, CHUNK // L, extract, 0)

        def fill_u(i, carry):
            out_slot[i, pl.ds(D_EMB, D_U)] = uvec
            return carry

        lax.fori_loop(0, CHUNK, fill_u, 0, unroll=8)

        writes.append(
            pltpu.async_copy(
                out_slot,
                out_hbm.at[pl.ds(base + j * CHUNK, CHUNK)],
                wsem,
            )
        )
        if j + NSLOT < NCHUNK:
            gathers.append(start_gather(j + NSLOT))

    for w in writes[-NSLOT:]:
        w.wait()


_emb_call = pl.kernel(
    _emb_body,
    mesh=plsc.VectorSubcoreMesh(core_axis_name="c", subcore_axis_name="s"),
    out_type=jax.ShapeDtypeStruct((B, D_OUT), jnp.float32),
    compiler_params=pltpu.CompilerParams(needs_layout_passes=False),
    scratch_types=[
        pltpu.VMEM((NCHUNK, CHUNK), jnp.int32),
        pltpu.VMEM((NCHUNK, CHUNK), jnp.int32),
        pltpu.VMEM((NSLOT, CHUNK, D_PACK), jnp.float32),
        pltpu.VMEM((NSLOT, CHUNK, D_OUT), jnp.float32),
        pltpu.VMEM((D_U,), jnp.float32),
        pltpu.SemaphoreType.DMA,
        pltpu.SemaphoreType.DMA,
        pltpu.SemaphoreType.DMA,
    ],
)


def kernel(x, table, unique):
    return _emb_call(x, table.reshape(NUM_PACKED, D_PACK), unique)


# speedup vs baseline: 1.4900x; 1.3425x over previous
import jax
import jax.numpy as jnp
from jax import lax
from jax.experimental import pallas as pl
from jax.experimental.pallas import tpu as pltpu
from jax.experimental.pallas import tpu_sc as plsc

B = 16384
NUM_ROWS = 100000
D_EMB = 48
D_U = 16
D_OUT = D_EMB + D_U
D_PAD = 128
L = 16

NC = 2
NS = 16
NW = NC * NS
BPW = B // NW
CHUNK = 128
NCHUNK = BPW // CHUNK
NSLOT = 2


def _emb_body(x_hbm, table_hbm, unique_hbm, out_hbm, idx_v, rows_v, out_v,
              u16_v, gsem, usem, wsem):
    wid = lax.axis_index("s") * NC + lax.axis_index("c")
    base = wid * BPW

    for j in range(NCHUNK):
        pltpu.sync_copy(x_hbm.at[pl.ds(base + j * CHUNK, CHUNK)], idx_v.at[j])

    pltpu.async_copy(unique_hbm, u16_v, usem).wait()
    uvec = u16_v[...]

    def start_gather(j):
        return pltpu.async_copy(
            table_hbm.at[idx_v.at[j]], rows_v.at[j % NSLOT], gsem
        )

    gathers = [start_gather(0), start_gather(1)]
    writes = []
    for j in range(NCHUNK):
        slot = j % NSLOT
        gathers[j].wait()

        def interleave(i, carry):
            a = rows_v[slot, i, pl.ds(0, L)]
            b = rows_v[slot, i, pl.ds(L, L)]
            c = rows_v[slot, i, pl.ds(2 * L, L)]
            out_v[slot, i, pl.ds(0, L)] = a
            out_v[slot, i, pl.ds(L, L)] = b
            out_v[slot, i, pl.ds(2 * L, L)] = c
            out_v[slot, i, pl.ds(3 * L, L)] = uvec
            return carry

        if j >= NSLOT:
            writes[j - NSLOT].wait()
        lax.fori_loop(0, CHUNK, interleave, 0, unroll=8)
        writes.append(
            pltpu.async_copy(
                out_v.at[slot],
                out_hbm.at[pl.ds(base + j * CHUNK, CHUNK)],
                wsem,
            )
        )
        if j + NSLOT < NCHUNK:
            gathers.append(start_gather(j + NSLOT))

    for w in writes[-NSLOT:]:
        w.wait()


_emb_call = pl.kernel(
    _emb_body,
    mesh=plsc.VectorSubcoreMesh(core_axis_name="c", subcore_axis_name="s"),
    out_type=jax.ShapeDtypeStruct((B, D_OUT), jnp.float32),
    scratch_types=[
        pltpu.VMEM((NCHUNK, CHUNK), jnp.int32),
        pltpu.VMEM((NSLOT, CHUNK, D_PAD), jnp.float32),
        pltpu.VMEM((NSLOT, CHUNK, D_OUT), jnp.float32),
        pltpu.VMEM((D_U,), jnp.float32),
        pltpu.SemaphoreType.DMA,
        pltpu.SemaphoreType.DMA,
        pltpu.SemaphoreType.DMA,
    ],
)


def kernel(x, table, unique):
    table_p = jnp.pad(table, ((0, 0), (0, D_PAD - D_EMB)))
    return _emb_call(x, table_p, unique)
